# baseline (speedup 1.0000x reference)
import functools

import jax
import jax.numpy as jnp
from jax import lax
from jax.experimental import pallas as pl
from jax.experimental.pallas import tpu as pltpu
from jax.experimental.pallas import tpu_sc as plsc

_EMBED = 32
_NC = 2
_NS = 16
_NW = _NC * _NS

_B_TOTAL = 4096 * 50
_B_PER_W = _B_TOTAL // _NW
_CHUNK = 1600
_NCHUNK = _B_PER_W // _CHUNK

_VOCAB = 1000000
_TW = 768
_TMAIN = 999936
_TBLK = _TMAIN // _TW
_TAIL = _VOCAB - _TMAIN


def _transpose_body(tT_hbm, tlin_hbm, b0, b1, o0, o1, si0, si1, so0, so1):
    wid = lax.axis_index("s") * _NC + lax.axis_index("c")
    lanes = lax.broadcasted_iota(jnp.int32, (16,), 0)
    k2 = lanes * _EMBED

    def fire_in(slot, buf, sem):
        blk = slot * _NW + wid

        @pl.when(blk < _TBLK)
        def _():
            pltpu.async_copy(
                tT_hbm.at[:, pl.ds(blk * _TW, _TW)],
                buf.at[:, pl.ds(0, _TW)], sem,
            )

    def wait_in(slot, buf, sem):
        blk = slot * _NW + wid

        @pl.when(blk < _TBLK)
        def _():
            pltpu.make_async_copy(
                tT_hbm.at[:, pl.ds(0, _TW)], buf.at[:, pl.ds(0, _TW)], sem
            ).wait()

    def fire_out(slot, out, sem):
        blk = slot * _NW + wid

        @pl.when(blk < _TBLK)
        def _():
            pltpu.async_copy(
                out.at[pl.ds(0, _TW * _EMBED)],
                tlin_hbm.at[pl.ds(blk * (_TW * _EMBED), _TW * _EMBED)],
                sem,
            )

    def drain_out(slot, out, sem):
        blk = slot * _NW + wid

        @pl.when(blk < _TBLK)
        def _():
            pltpu.make_async_copy(
                out.at[pl.ds(0, _TW * _EMBED)],
                tlin_hbm.at[pl.ds(0, _TW * _EMBED)], sem
            ).wait()

    def shuffle(slot, buf, out):
        blk = slot * _NW + wid

        @pl.when(blk < _TBLK)
        def _():
            c_lo = lanes
            c_hi = lanes + 16

            def g_body(g, _):
                vs = []
                for k in range(8):
                    r = jnp.full((16,), g * 8 + k, jnp.int32)
                    vs.append(plsc.load_gather(buf, [c_lo, r]))
                    vs.append(plsc.load_gather(buf, [c_hi, r]))
                for k in range(8):
                    off = (g * 8 + k) * _EMBED
                    out[pl.ds(off, 16)] = vs[2 * k]
                    out[pl.ds(off + 16, 16)] = vs[2 * k + 1]
                return 0

            lax.fori_loop(0, _TW // 8, g_body, 0)

    fire_in(0, b0, si0)
    fire_in(1, b1, si1)

    def pair_body(bp, _):
        i0 = 2 * bp
        i1 = 2 * bp + 1
        wait_in(i0, b0, si0)

        @pl.when(bp > 0)
        def _():
            drain_out(i0 - 2, o0, so0)

        shuffle(i0, b0, o0)
        fire_out(i0, o0, so0)
        fire_in(i0 + 2, b0, si0)

        wait_in(i1, b1, si1)

        @pl.when(bp > 0)
        def _():
            drain_out(i1 - 2, o1, so1)

        shuffle(i1, b1, o1)
        fire_out(i1, o1, so1)
        fire_in(i1 + 2, b1, si1)
        return 0

    lax.fori_loop(0, 21, pair_body, 0)
    drain_out(42, o0, so0)
    drain_out(43, o1, so1)


def _emb_body(idx_hbm, table_hbm, tail_hbm, out_hbm, idx_v, idx2_v, rows_v,
              tail_v, sem):
    wid = lax.axis_index("s") * _NC + lax.axis_index("c")
    lanes = lax.broadcasted_iota(jnp.int32, (16,), 0)
    base = wid * _B_PER_W
    pltpu.sync_copy(idx_hbm.at[pl.ds(base, _B_PER_W)], idx_v)
    pltpu.sync_copy(tail_hbm, tail_v)

    def clamp_body(g, _):
        v = idx_v[pl.ds(g * 16, 16)]
        idx2_v[pl.ds(g * 16, 16)] = jnp.minimum(v, _TMAIN - 1)
        return 0

    lax.fori_loop(0, _B_PER_W // 16, clamp_body, 0)

    for c in range(_NCHUNK):
        off = c * _CHUNK
        pltpu.async_copy(
            table_hbm.at[idx2_v.at[pl.ds(off, _CHUNK)]], rows_v, sem
        ).wait()

        def fix_body(g, _):
            v = idx_v[pl.ds(off + g * 16, 16)]
            m = v >= _TMAIN
            cnt = plsc.all_reduce_population_count(m)

            @pl.when(cnt[0] > 0)
            def _():
                vt = jnp.maximum(v - _TMAIN, 0)
                row = lanes + g * 16
                for e in range(_EMBED):
                    col = jnp.full((16,), e, jnp.int32)
                    vals = plsc.load_gather(tail_v, [vt, col], mask=m)
                    plsc.store_scatter(rows_v, [row, col], vals, mask=m)
            return 0

        lax.fori_loop(0, _CHUNK // 16, fix_body, 0)
        pltpu.sync_copy(rows_v, out_hbm.at[pl.ds(base + off, _CHUNK)])


@jax.jit
def kernel(tokens_inputs, table):
    idx = tokens_inputs.reshape(-1)
    tail = table[_TMAIN:]
    mesh = plsc.VectorSubcoreMesh(core_axis_name="c", subcore_axis_name="s")

    transpose_fn = functools.partial(
        pl.kernel,
        out_type=jax.ShapeDtypeStruct((_TMAIN * _EMBED,), jnp.float32),
        mesh=mesh,
        scratch_types=[
            pltpu.VMEM((_EMBED, _TW + 1), jnp.float32),
            pltpu.VMEM((_EMBED, _TW + 1), jnp.float32),
            pltpu.VMEM((_TW * _EMBED + 32,), jnp.float32),
            pltpu.VMEM((_TW * _EMBED + 32,), jnp.float32),
            pltpu.SemaphoreType.DMA,
            pltpu.SemaphoreType.DMA,
            pltpu.SemaphoreType.DMA,
            pltpu.SemaphoreType.DMA,
        ],
        compiler_params=pltpu.CompilerParams(needs_layout_passes=False),
    )(_transpose_body)

    tlin = transpose_fn(table.T)
    table2d = tlin.reshape(_TMAIN, _EMBED)

    gather_fn = functools.partial(
        pl.kernel,
        out_type=jax.ShapeDtypeStruct((_B_TOTAL, _EMBED), jnp.float32),
        mesh=mesh,
        scratch_types=[
            pltpu.VMEM((_B_PER_W,), jnp.int32),
            pltpu.VMEM((_B_PER_W,), jnp.int32),
            pltpu.VMEM((_CHUNK, _EMBED), jnp.float32),
            pltpu.VMEM((_TAIL, _EMBED), jnp.float32),
            pltpu.SemaphoreType.DMA,
        ],
        compiler_params=pltpu.CompilerParams(
            use_tc_tiling_on_sc=False, needs_layout_passes=False
        ),
    )(_emb_body)
    out = gather_fn(idx, table2d, tail)
    outb = lax.optimization_barrier(out.reshape(51200, 128))
    return outb.reshape(4096, 50, _EMBED)


# speedup vs baseline: 1.0261x; 1.0261x over previous
import functools

import jax
import jax.numpy as jnp
from jax import lax
from jax.experimental import pallas as pl
from jax.experimental.pallas import tpu as pltpu
from jax.experimental.pallas import tpu_sc as plsc

_EMBED = 32
_NC = 2
_NS = 16
_NW = _NC * _NS

_B_TOTAL = 4096 * 50
_B_PER_W = _B_TOTAL // _NW
_CHUNK = 1600
_NCHUNK = _B_PER_W // _CHUNK

_VOCAB = 1000000
_TW = 768
_TMAIN = 999936
_TBLK = _TMAIN // _TW
_TAIL = _VOCAB - _TMAIN


def _transpose_body(tT_hbm, tlin_hbm, b0, b1, o0, o1, si0, si1, so0, so1):
    wid = lax.axis_index("s") * _NC + lax.axis_index("c")
    lanes = lax.broadcasted_iota(jnp.int32, (16,), 0)
    k2 = lanes * _EMBED

    def fire_in(slot, buf, sem):
        blk = slot * _NW + wid

        @pl.when(blk < _TBLK)
        def _():
            pltpu.async_copy(
                tT_hbm.at[:, pl.ds(blk * _TW, _TW)],
                buf.at[:, pl.ds(0, _TW)], sem,
            )

    def wait_in(slot, buf, sem):
        blk = slot * _NW + wid

        @pl.when(blk < _TBLK)
        def _():
            pltpu.make_async_copy(
                tT_hbm.at[:, pl.ds(0, _TW)], buf.at[:, pl.ds(0, _TW)], sem
            ).wait()

    def fire_out(slot, out, sem):
        blk = slot * _NW + wid

        @pl.when(blk < _TBLK)
        def _():
            pltpu.async_copy(
                out.at[pl.ds(0, _TW * _EMBED)],
                tlin_hbm.at[pl.ds(blk * (_TW * _EMBED), _TW * _EMBED)],
                sem,
            )

    def drain_out(slot, out, sem):
        blk = slot * _NW + wid

        @pl.when(blk < _TBLK)
        def _():
            pltpu.make_async_copy(
                out.at[pl.ds(0, _TW * _EMBED)],
                tlin_hbm.at[pl.ds(0, _TW * _EMBED)], sem
            ).wait()

    def shuffle(slot, buf, out):
        blk = slot * _NW + wid

        @pl.when(blk < _TBLK)
        def _():
            def g_body(g, _):
                base = out.at[pl.ds(g * (16 * _EMBED), 16 * _EMBED)]
                for h in range(2):
                    cs = range(h * 16, h * 16 + 16)
                    vs = [buf[c, pl.ds(g * 16, 16)] for c in cs]
                    for j, c in enumerate(cs):
                        plsc.store_scatter(base, [k2 + c], vs[j])
                return 0

            lax.fori_loop(0, _TW // 16, g_body, 0)

    fire_in(0, b0, si0)
    fire_in(1, b1, si1)

    def pair_body(bp, _):
        i0 = 2 * bp
        i1 = 2 * bp + 1
        wait_in(i0, b0, si0)

        @pl.when(bp > 0)
        def _():
            drain_out(i0 - 2, o0, so0)

        shuffle(i0, b0, o0)
        fire_out(i0, o0, so0)
        fire_in(i0 + 2, b0, si0)

        wait_in(i1, b1, si1)

        @pl.when(bp > 0)
        def _():
            drain_out(i1 - 2, o1, so1)

        shuffle(i1, b1, o1)
        fire_out(i1, o1, so1)
        fire_in(i1 + 2, b1, si1)
        return 0

    lax.fori_loop(0, 21, pair_body, 0)
    drain_out(42, o0, so0)
    drain_out(43, o1, so1)


def _emb_body(idx_hbm, table_hbm, tail_hbm, out_hbm, idx_v, idx2_v, rows_v,
              tail_v, sem):
    wid = lax.axis_index("s") * _NC + lax.axis_index("c")
    lanes = lax.broadcasted_iota(jnp.int32, (16,), 0)
    base = wid * _B_PER_W
    pltpu.sync_copy(idx_hbm.at[pl.ds(base, _B_PER_W)], idx_v)
    pltpu.sync_copy(tail_hbm, tail_v)

    def clamp_body(g, _):
        v = idx_v[pl.ds(g * 16, 16)]
        idx2_v[pl.ds(g * 16, 16)] = jnp.minimum(v, _TMAIN - 1)
        return 0

    lax.fori_loop(0, _B_PER_W // 16, clamp_body, 0)

    for c in range(_NCHUNK):
        off = c * _CHUNK
        pltpu.async_copy(
            table_hbm.at[idx2_v.at[pl.ds(off, _CHUNK)]], rows_v, sem
        ).wait()

        def fix_body(g, _):
            v = idx_v[pl.ds(off + g * 16, 16)]
            m = v >= _TMAIN
            cnt = plsc.all_reduce_population_count(m)

            @pl.when(cnt[0] > 0)
            def _():
                vt = jnp.maximum(v - _TMAIN, 0)
                row = lanes + g * 16
                for e in range(_EMBED):
                    col = jnp.full((16,), e, jnp.int32)
                    vals = plsc.load_gather(tail_v, [vt, col], mask=m)
                    plsc.store_scatter(rows_v, [row, col], vals, mask=m)
            return 0

        lax.fori_loop(0, _CHUNK // 16, fix_body, 0)
        pltpu.sync_copy(rows_v, out_hbm.at[pl.ds(base + off, _CHUNK)])


@jax.jit
def kernel(tokens_inputs, table):
    idx = tokens_inputs.reshape(-1)
    tail = table[_TMAIN:]
    mesh = plsc.VectorSubcoreMesh(core_axis_name="c", subcore_axis_name="s")

    transpose_fn = functools.partial(
        pl.kernel,
        out_type=jax.ShapeDtypeStruct((_TMAIN * _EMBED,), jnp.float32),
        mesh=mesh,
        scratch_types=[
            pltpu.VMEM((_EMBED, _TW), jnp.float32),
            pltpu.VMEM((_EMBED, _TW), jnp.float32),
            pltpu.VMEM((_TW * _EMBED + 32,), jnp.float32),
            pltpu.VMEM((_TW * _EMBED + 32,), jnp.float32),
            pltpu.SemaphoreType.DMA,
            pltpu.SemaphoreType.DMA,
            pltpu.SemaphoreType.DMA,
            pltpu.SemaphoreType.DMA,
        ],
        compiler_params=pltpu.CompilerParams(needs_layout_passes=False),
    )(_transpose_body)

    tlin = transpose_fn(table.T)
    table2d = tlin.reshape(_TMAIN, _EMBED)

    gather_fn = functools.partial(
        pl.kernel,
        out_type=jax.ShapeDtypeStruct((_B_TOTAL, _EMBED), jnp.float32),
        mesh=mesh,
        scratch_types=[
            pltpu.VMEM((_B_PER_W,), jnp.int32),
            pltpu.VMEM((_B_PER_W,), jnp.int32),
            pltpu.VMEM((_CHUNK, _EMBED), jnp.float32),
            pltpu.VMEM((_TAIL, _EMBED), jnp.float32),
            pltpu.SemaphoreType.DMA,
        ],
        compiler_params=pltpu.CompilerParams(
            use_tc_tiling_on_sc=False, needs_layout_passes=False
        ),
    )(_emb_body)
    out = gather_fn(idx, table2d, tail)
    outb = lax.optimization_barrier(out.reshape(51200, 128))
    return outb.reshape(4096, 50, _EMBED)
